# Initial kernel scaffold; baseline (speedup 1.0000x reference)
import functools

import jax
import jax.numpy as jnp
from jax import lax
from jax.experimental import pallas as pl
from jax.experimental.pallas import tpu as pltpu
from jax.experimental.pallas import tpu_sc as plsc

N = 10000
E = 320000
D_IN = 128
D_H = 256
D_OUT = 64
K = 5
ALPHA = 0.1

NC = 2
NS = 16
NW = NC * NS
NPAD = 10240
RPS = NPAD // NS
EPAD = 327680
IW = 128
EROWS = EPAD // IW
RPW = EROWS // NW
NB = 8
OUTER = RPW // NB

_mesh = plsc.VectorSubcoreMesh(core_axis_name="c", subcore_axis_name="s")


@functools.partial(
    pl.kernel,
    out_type=jax.ShapeDtypeStruct((NC, NPAD, 16), jnp.float32),
    mesh=_mesh,
    scratch_types=[
        pltpu.VMEM((NB, IW), jnp.int32),
        pltpu.VMEM((IW, 16), jnp.float32),
        pltpu.VMEM_SHARED((NPAD, 16), jnp.float32),
    ],
)
def _deg_sc(dst_hbm, ones_hbm, zeros16_hbm, out_hbm, dstv, onev, acc_sh):
    cid = lax.axis_index("c")
    sid = lax.axis_index("s")
    wid = cid * NS + sid
    pltpu.sync_copy(zeros16_hbm, acc_sh.at[pl.ds(sid * RPS, RPS)])
    pltpu.sync_copy(ones_hbm, onev)
    plsc.subcore_barrier()

    @pl.loop(0, OUTER)
    def _(it):
        base = wid * RPW + it * NB
        pltpu.sync_copy(dst_hbm.at[pl.ds(base, NB)], dstv)
        for j in range(NB):
            pltpu.sync_copy(onev, acc_sh.at[dstv.at[j]], add=True)

    plsc.subcore_barrier()
    pltpu.sync_copy(acc_sh.at[pl.ds(sid * RPS, RPS)],
                    out_hbm.at[cid].at[pl.ds(sid * RPS, RPS)])


@functools.partial(
    pl.kernel,
    out_type=jax.ShapeDtypeStruct((NC, NPAD, D_OUT), jnp.float32),
    mesh=_mesh,
    scratch_types=[
        pltpu.VMEM((NB, IW), jnp.int32),
        pltpu.VMEM((NB, IW), jnp.int32),
        pltpu.VMEM((NB * IW, D_OUT), jnp.float32),
        pltpu.VMEM_SHARED((NPAD, D_OUT), jnp.float32),
        pltpu.SemaphoreType.DMA,
    ],
)
def _prop_sc(table_hbm, src_hbm, dst_hbm, zeros_hbm, out_hbm,
             srcv, dstv, rows, acc_sh, sem):
    cid = lax.axis_index("c")
    sid = lax.axis_index("s")
    wid = cid * NS + sid
    pltpu.sync_copy(zeros_hbm, acc_sh.at[pl.ds(sid * RPS, RPS)])
    plsc.subcore_barrier()

    @pl.loop(0, OUTER)
    def _(it):
        base = wid * RPW + it * NB
        pltpu.sync_copy(src_hbm.at[pl.ds(base, NB)], srcv)
        pltpu.sync_copy(dst_hbm.at[pl.ds(base, NB)], dstv)
        copies = []
        for j in range(NB):
            copies.append(pltpu.async_copy(
                table_hbm.at[srcv.at[j]],
                rows.at[pl.ds(j * IW, IW)], sem))
        for j in range(NB):
            copies[j].wait()
            pltpu.sync_copy(rows.at[pl.ds(j * IW, IW)],
                            acc_sh.at[dstv.at[j]], add=True)

    plsc.subcore_barrier()
    pltpu.sync_copy(acc_sh.at[pl.ds(sid * RPS, RPS)],
                    out_hbm.at[cid].at[pl.ds(sid * RPS, RPS)])


def _mlp_body(x_ref, w1_ref, b1_ref, w2_ref, b2_ref, h0_ref):
    h = jnp.dot(x_ref[...], w1_ref[...], preferred_element_type=jnp.float32,
                precision=lax.Precision.HIGHEST)
    h = jnp.maximum(h + b1_ref[...], 0.0)
    h0_ref[...] = jnp.dot(h, w2_ref[...], preferred_element_type=jnp.float32,
                          precision=lax.Precision.HIGHEST) + b2_ref[...]


def _prep_body(h0_ref, dacc_ref, g0_ref, cb_ref, sd_ref):
    deg = dacc_ref[0, :, 0:1] + dacc_ref[1, :, 0:1] + 1.0
    rs = lax.rsqrt(deg)
    g0_ref[...] = rs * h0_ref[...]
    cb_ref[...] = jnp.broadcast_to((1.0 - ALPHA) / deg, (NPAD, D_OUT))
    sd_ref[...] = jnp.broadcast_to(jnp.sqrt(deg), (NPAD, D_OUT))


def _combine_body(acc_ref, cb_ref, g0_ref, g_ref):
    g_ref[...] = cb_ref[...] * (acc_ref[0] + acc_ref[1]) + ALPHA * g0_ref[...]


def _final_body(acc_ref, cb_ref, g0_ref, sd_ref, o_ref):
    h = (cb_ref[...] * (acc_ref[0] + acc_ref[1]) + ALPHA * g0_ref[...]) * sd_ref[...]
    m = jnp.max(h, axis=1, keepdims=True)
    e = jnp.exp(h - m)
    o_ref[...] = h - m - jnp.log(jnp.sum(e, axis=1, keepdims=True))


_mlp_tc = pl.pallas_call(
    _mlp_body,
    out_shape=jax.ShapeDtypeStruct((NPAD, D_OUT), jnp.float32),
)

_prep_tc = pl.pallas_call(
    _prep_body,
    out_shape=(
        jax.ShapeDtypeStruct((NPAD, D_OUT), jnp.float32),
        jax.ShapeDtypeStruct((NPAD, D_OUT), jnp.float32),
        jax.ShapeDtypeStruct((NPAD, D_OUT), jnp.float32),
    ),
)

_combine_tc = pl.pallas_call(
    _combine_body,
    out_shape=jax.ShapeDtypeStruct((NPAD, D_OUT), jnp.float32),
)

_final_tc = pl.pallas_call(
    _final_body,
    out_shape=jax.ShapeDtypeStruct((NPAD, D_OUT), jnp.float32),
)


def kernel(x, edge_index, W1, b1, W2, b2):
    x = x.astype(jnp.float32)
    src = edge_index[0]
    dst = edge_index[1]
    npad_e = EPAD - E
    pad_ids = (N + (jnp.arange(npad_e, dtype=jnp.int32) % (NPAD - N)))
    src2d = jnp.concatenate([src, pad_ids]).reshape(EROWS, IW)
    dst2d = jnp.concatenate([dst, pad_ids]).reshape(EROWS, IW)

    xp = jnp.pad(x, ((0, NPAD - N), (0, 0)))
    ones16 = jnp.ones((IW, 16), jnp.float32)
    zeros16 = jnp.zeros((RPS, 16), jnp.float32)
    zeros64 = jnp.zeros((RPS, D_OUT), jnp.float32)

    dacc = _deg_sc(dst2d, ones16, zeros16)
    h0 = _mlp_tc(xp, W1, b1.reshape(1, D_H), W2, b2.reshape(1, D_OUT))
    g0, cb, sd = _prep_tc(h0, dacc)

    g = g0
    for k in range(K):
        acc = _prop_sc(g, src2d, dst2d, zeros64)
        if k < K - 1:
            g = _combine_tc(acc, cb, g0)
    out = _final_tc(acc, cb, g0, sd)
    return out[:N]


# speedup vs baseline: 8.6707x; 8.6707x over previous
import dataclasses
import functools

import jax
import jax.numpy as jnp
from jax import lax
from jax.experimental import pallas as pl
from jax.experimental.pallas import tpu as pltpu
from jax.experimental.pallas import tpu_sc as plsc

N = 10000
E = 320000
D_IN = 128
D_H = 256
D_OUT = 64
K = 5
ALPHA = 0.1

W = 128
NC = 2
NS = 16
NW = NC * NS
NPAD = 10240
RPS = NPAD // NS
EPAD = 327680
IW = 128
EROWS = EPAD // IW
RPW = EROWS // NW
NB = 8
HB = 4
OUTER = RPW // NB

_mesh = plsc.VectorSubcoreMesh(core_axis_name="c", subcore_axis_name="s")

_sc_params = pltpu.CompilerParams()
if "needs_layout_passes" in pltpu.CompilerParams.__dataclass_fields__:
    _sc_params = dataclasses.replace(_sc_params, needs_layout_passes=False)


ZR = 64


def _fill(buf, nrows, value):
    vec = jnp.full((16,), value, jnp.float32)

    @pl.loop(0, nrows)
    def _(i):
        for c in range(W // 16):
            buf[i, pl.ds(c * 16, 16)] = vec


@functools.partial(
    pl.kernel,
    out_type=jax.ShapeDtypeStruct((NW, NPAD), jnp.float32),
    mesh=_mesh,
    compiler_params=_sc_params,
    scratch_types=[
        pltpu.VMEM((NB, IW), jnp.int32),
        pltpu.VMEM((NPAD,), jnp.float32),
    ],
)
def _deg_sc(dst_hbm, out_hbm, dstv, hist):
    cid = lax.axis_index("c")
    sid = lax.axis_index("s")
    wid = cid * NS + sid
    zvec = jnp.zeros((16,), jnp.float32)

    @pl.loop(0, NPAD // 16)
    def _(i):
        hist[pl.ds(i * 16, 16)] = zvec

    ovec = jnp.ones((16,), jnp.float32)

    @pl.loop(0, OUTER)
    def _(it):
        base = wid * RPW + it * NB
        pltpu.sync_copy(dst_hbm.at[pl.ds(base, NB)], dstv)
        for j in range(NB):
            for c in range(IW // 16):
                idx = dstv[j, pl.ds(c * 16, 16)]
                plsc.addupdate_scatter(hist, [idx], ovec)

    pltpu.sync_copy(hist, out_hbm.at[wid])


NHALF = NPAD // NC
TRASH = 512
AROWS = NHALF + TRASH
ZPS = AROWS // NS
CRPS = NHALF // NS
RPS2 = EROWS // NS
OUT2 = RPS2 // NB


@functools.partial(
    pl.kernel,
    out_type=jax.ShapeDtypeStruct((NC, NHALF, W), jnp.float32),
    mesh=_mesh,
    compiler_params=_sc_params,
    scratch_types=[
        pltpu.VMEM((NB, IW), jnp.int32),
        pltpu.VMEM((NB, IW), jnp.int32),
        pltpu.VMEM((NB, IW), jnp.int32),
        pltpu.VMEM((HB * IW, W), jnp.float32),
        pltpu.VMEM((ZR, W), jnp.float32),
        pltpu.VMEM_SHARED((AROWS, W), jnp.float32),
        pltpu.SemaphoreType.DMA,
    ],
)
def _prop_sc(table_hbm, src_hbm, dst_hbm, out_hbm,
             srcv, dstv, dstw, rows, zbuf, acc_sh, sem):
    cid = lax.axis_index("c")
    sid = lax.axis_index("s")
    _fill(zbuf, ZR, 0.0)
    for t in range(ZPS // ZR):
        pltpu.sync_copy(zbuf, acc_sh.at[pl.ds(sid * ZPS + t * ZR, ZR)])
    for t in range(ZPS % ZR // 16):
        pltpu.sync_copy(zbuf.at[pl.ds(0, 16)],
                        acc_sh.at[pl.ds(sid * ZPS + (ZPS // ZR) * ZR + t * 16, 16)])
    plsc.subcore_barrier()

    lo = jnp.broadcast_to(cid * NHALF, (16,)).astype(jnp.int32)
    hi = lo + NHALF
    tbase = jnp.broadcast_to(NHALF, (16,)).astype(jnp.int32)
    tmask = jnp.broadcast_to(TRASH - 1, (16,)).astype(jnp.int32)

    @pl.loop(0, OUT2)
    def _(it):
        base = sid * RPS2 + it * NB
        pltpu.sync_copy(src_hbm.at[pl.ds(base, NB)], srcv)
        pltpu.sync_copy(dst_hbm.at[pl.ds(base, NB)], dstv)
        for j in range(NB):
            for c in range(IW // 16):
                d = dstv[j, pl.ds(c * 16, 16)]
                mine = (d >= lo) & (d < hi)
                dloc = d - lo
                dtrash = tbase + (d & tmask)
                dstw[j, pl.ds(c * 16, 16)] = jnp.where(mine, dloc, dtrash)
        for half in range(NB // HB):
            copies = []
            for j in range(HB):
                copies.append(pltpu.async_copy(
                    table_hbm.at[srcv.at[half * HB + j]],
                    rows.at[pl.ds(j * IW, IW)], sem))
            for j in range(HB):
                copies[j].wait()
                pltpu.sync_copy(rows.at[pl.ds(j * IW, IW)],
                                acc_sh.at[dstw.at[half * HB + j]], add=True)

    plsc.subcore_barrier()
    pltpu.sync_copy(acc_sh.at[pl.ds(sid * CRPS, CRPS)],
                    out_hbm.at[cid].at[pl.ds(sid * CRPS, CRPS)])


def _mlp_body(x_ref, w1_ref, b1_ref, w2_ref, b2_ref, h0_ref):
    h = jnp.dot(x_ref[...], w1_ref[...], preferred_element_type=jnp.float32,
                precision=lax.Precision.HIGHEST)
    h = jnp.maximum(h + b1_ref[...], 0.0)
    h0_ref[...] = jnp.dot(h, w2_ref[...], preferred_element_type=jnp.float32,
                          precision=lax.Precision.HIGHEST) + b2_ref[...]


def _prep_body(h0_ref, dacc_ref, g0_ref, cb_ref, sd_ref):
    deg = lax.dot_general(
        dacc_ref[...], jnp.ones((NW, 1), jnp.float32),
        dimension_numbers=(((0,), (0,)), ((), ())),
        preferred_element_type=jnp.float32) + 1.0
    rs = lax.rsqrt(deg)
    g0_ref[...] = rs * h0_ref[...]
    cb_ref[...] = jnp.broadcast_to((1.0 - ALPHA) / deg, (NPAD, W))
    sd_ref[...] = jnp.broadcast_to(jnp.sqrt(deg), (NPAD, W))


def _combine_body(acc_ref, cb_ref, g0_ref, g_ref):
    g_ref[...] = cb_ref[...] * acc_ref[...] + ALPHA * g0_ref[...]


def _final_body(g_ref, sd_ref, o_ref):
    h = (g_ref[...] * sd_ref[...])[:, :D_OUT]
    m = jnp.max(h, axis=1, keepdims=True)
    e = jnp.exp(h - m)
    o_ref[...] = h - m - jnp.log(jnp.sum(e, axis=1, keepdims=True))


_mlp_tc = pl.pallas_call(
    _mlp_body,
    out_shape=jax.ShapeDtypeStruct((NPAD, W), jnp.float32),
)

_prep_tc = pl.pallas_call(
    _prep_body,
    out_shape=(
        jax.ShapeDtypeStruct((NPAD, W), jnp.float32),
        jax.ShapeDtypeStruct((NPAD, W), jnp.float32),
        jax.ShapeDtypeStruct((NPAD, W), jnp.float32),
    ),
)

_combine_tc = pl.pallas_call(
    _combine_body,
    out_shape=jax.ShapeDtypeStruct((NPAD, W), jnp.float32),
)

_final_tc = pl.pallas_call(
    _final_body,
    out_shape=jax.ShapeDtypeStruct((NPAD, D_OUT), jnp.float32),
)


def kernel(x, edge_index, W1, b1, W2, b2):
    x = x.astype(jnp.float32)
    src = edge_index[0]
    dst = edge_index[1]
    npad_e = EPAD - E
    pad_ids = (N + (jnp.arange(npad_e, dtype=jnp.int32) % (NPAD - N)))
    src2d = jnp.concatenate([src, pad_ids]).reshape(EROWS, IW)
    dst2d = jnp.concatenate([dst, pad_ids]).reshape(EROWS, IW)

    xp = jnp.pad(x, ((0, NPAD - N), (0, 0)))
    w2p = jnp.pad(W2, ((0, 0), (0, W - D_OUT)))
    b2p = jnp.pad(b2, ((0, W - D_OUT),)).reshape(1, W)
    dacc = _deg_sc(dst2d)
    h0 = _mlp_tc(xp, W1, b1.reshape(1, D_H), w2p, b2p)
    g0, cb, sd = _prep_tc(h0, dacc)

    def _round(_, g):
        acc = _prop_sc(g, src2d, dst2d)
        return _combine_tc(acc.reshape(NPAD, W), cb, g0)

    g = lax.fori_loop(0, K, _round, g0)
    out = _final_tc(g, sd)
    return out[:N]


# speedup vs baseline: 9.6502x; 1.1130x over previous
import dataclasses
import functools

import jax
import jax.numpy as jnp
from jax import lax
from jax.experimental import pallas as pl
from jax.experimental.pallas import tpu as pltpu
from jax.experimental.pallas import tpu_sc as plsc

N = 10000
E = 320000
D_IN = 128
D_H = 256
D_OUT = 64
K = 5
ALPHA = 0.1

W = 128
NC = 2
NS = 16
NW = NC * NS
NPAD = 10240
RPS = NPAD // NS
EPAD = 327680
IW = 128
EROWS = EPAD // IW
RPW = EROWS // NW
NB = 8
HB = 4
OUTER = RPW // NB

_mesh = plsc.VectorSubcoreMesh(core_axis_name="c", subcore_axis_name="s")

_sc_params = pltpu.CompilerParams()
if "needs_layout_passes" in pltpu.CompilerParams.__dataclass_fields__:
    _sc_params = dataclasses.replace(_sc_params, needs_layout_passes=False)


ZR = 64


def _fill(buf, nrows, value):
    vec = jnp.full((16,), value, jnp.float32)

    @pl.loop(0, nrows)
    def _(i):
        for c in range(W // 16):
            buf[i, pl.ds(c * 16, 16)] = vec


@functools.partial(
    pl.kernel,
    out_type=jax.ShapeDtypeStruct((NW, NPAD), jnp.float32),
    mesh=_mesh,
    compiler_params=_sc_params,
    scratch_types=[
        pltpu.VMEM((NB, IW), jnp.int32),
        pltpu.VMEM((NPAD,), jnp.float32),
    ],
)
def _deg_sc(dst_hbm, out_hbm, dstv, hist):
    cid = lax.axis_index("c")
    sid = lax.axis_index("s")
    wid = cid * NS + sid
    zvec = jnp.zeros((16,), jnp.float32)

    @pl.loop(0, NPAD // 16)
    def _(i):
        hist[pl.ds(i * 16, 16)] = zvec

    ovec = jnp.ones((16,), jnp.float32)

    @pl.loop(0, OUTER)
    def _(it):
        base = wid * RPW + it * NB
        pltpu.sync_copy(dst_hbm.at[pl.ds(base, NB)], dstv)
        for j in range(NB):
            for c in range(IW // 16):
                idx = dstv[j, pl.ds(c * 16, 16)]
                plsc.addupdate_scatter(hist, [idx], ovec)

    pltpu.sync_copy(hist, out_hbm.at[wid])


NHALF = NPAD // NC
TRASH = 512
AROWS = NHALF + TRASH
ZPS = AROWS // NS
CRPS = NHALF // NS
RPS2 = EROWS // NS
OUT2 = RPS2 // NB


NB2 = 8
OUT3 = RPS2 // NB2
SLOTS = 4
DEPTH = 3


@functools.partial(
    pl.kernel,
    out_type=jax.ShapeDtypeStruct((NC, NHALF, W), jnp.float32),
    mesh=_mesh,
    compiler_params=_sc_params,
    scratch_types=[
        pltpu.VMEM((NB2, IW), jnp.int32),
        pltpu.VMEM((NB2, IW), jnp.int32),
        pltpu.VMEM((NB2, IW), jnp.int32),
        pltpu.VMEM((SLOTS * IW, W), jnp.float32),
        pltpu.VMEM_SHARED((AROWS, W), jnp.float32),
        pltpu.SemaphoreType.DMA,
        pltpu.SemaphoreType.DMA,
    ],
)
def _prop_sc(table_hbm, src_hbm, dst_hbm, out_hbm,
             srcv, dstv, dstw, rows, acc_sh, gsem, ssem):
    cid = lax.axis_index("c")
    sid = lax.axis_index("s")
    _fill(rows, ZR, 0.0)
    for t in range(ZPS // ZR):
        pltpu.sync_copy(rows.at[pl.ds(0, ZR)],
                        acc_sh.at[pl.ds(sid * ZPS + t * ZR, ZR)])
    for t in range(ZPS % ZR // 16):
        pltpu.sync_copy(rows.at[pl.ds(0, 16)],
                        acc_sh.at[pl.ds(sid * ZPS + (ZPS // ZR) * ZR + t * 16, 16)])
    plsc.subcore_barrier()

    lo = jnp.broadcast_to(cid * NHALF, (16,)).astype(jnp.int32)
    hi = lo + NHALF
    tbase = jnp.broadcast_to(NHALF, (16,)).astype(jnp.int32)
    tmask = jnp.broadcast_to(TRASH - 1, (16,)).astype(jnp.int32)

    @pl.loop(0, OUT3)
    def _(it):
        base = sid * RPS2 + it * NB2
        pltpu.sync_copy(src_hbm.at[pl.ds(base, NB2)], srcv)
        pltpu.sync_copy(dst_hbm.at[pl.ds(base, NB2)], dstv)
        for j in range(NB2):
            for c in range(IW // 16):
                d = dstv[j, pl.ds(c * 16, 16)]
                mine = (d >= lo) & (d < hi)
                dstw[j, pl.ds(c * 16, 16)] = jnp.where(
                    mine, d - lo, tbase + (d & tmask))

        def _gather(j):
            s = j % SLOTS
            return pltpu.async_copy(
                table_hbm.at[srcv.at[j]], rows.at[pl.ds(s * IW, IW)], gsem)

        gh = [None] * SLOTS
        sh = [None] * SLOTS
        for j in range(DEPTH):
            gh[j % SLOTS] = _gather(j)
        for j in range(NB2):
            s = j % SLOTS
            gh[s].wait()
            sh[s] = pltpu.async_copy(
                rows.at[pl.ds(s * IW, IW)], acc_sh.at[dstw.at[j]], ssem,
                add=True)
            nj = j + DEPTH
            if nj < NB2:
                s2 = nj % SLOTS
                if sh[s2] is not None:
                    sh[s2].wait()
                    sh[s2] = None
                gh[s2] = _gather(nj)
        for s in range(SLOTS):
            if sh[s] is not None:
                sh[s].wait()
                sh[s] = None

    plsc.subcore_barrier()
    pltpu.sync_copy(acc_sh.at[pl.ds(sid * CRPS, CRPS)],
                    out_hbm.at[cid].at[pl.ds(sid * CRPS, CRPS)])


def _mlp_body(x_ref, w1_ref, b1_ref, w2_ref, b2_ref, h0_ref):
    h = jnp.dot(x_ref[...], w1_ref[...], preferred_element_type=jnp.float32,
                precision=lax.Precision.HIGHEST)
    h = jnp.maximum(h + b1_ref[...], 0.0)
    h0_ref[...] = jnp.dot(h, w2_ref[...], preferred_element_type=jnp.float32,
                          precision=lax.Precision.HIGHEST) + b2_ref[...]


def _prep_body(h0_ref, dacc_ref, g0_ref, cb_ref, sd_ref):
    deg = lax.dot_general(
        dacc_ref[...], jnp.ones((NW, 1), jnp.float32),
        dimension_numbers=(((0,), (0,)), ((), ())),
        preferred_element_type=jnp.float32) + 1.0
    rs = lax.rsqrt(deg)
    g0_ref[...] = rs * h0_ref[...]
    cb_ref[...] = jnp.broadcast_to((1.0 - ALPHA) / deg, (NPAD, W))
    sd_ref[...] = jnp.broadcast_to(jnp.sqrt(deg), (NPAD, W))


def _combine_body(acc_ref, cb_ref, g0_ref, g_ref):
    g_ref[...] = cb_ref[...] * acc_ref[...] + ALPHA * g0_ref[...]


def _final_body(g_ref, sd_ref, o_ref):
    h = (g_ref[...] * sd_ref[...])[:, :D_OUT]
    m = jnp.max(h, axis=1, keepdims=True)
    e = jnp.exp(h - m)
    o_ref[...] = h - m - jnp.log(jnp.sum(e, axis=1, keepdims=True))


_mlp_tc = pl.pallas_call(
    _mlp_body,
    out_shape=jax.ShapeDtypeStruct((NPAD, W), jnp.float32),
)

_prep_tc = pl.pallas_call(
    _prep_body,
    out_shape=(
        jax.ShapeDtypeStruct((NPAD, W), jnp.float32),
        jax.ShapeDtypeStruct((NPAD, W), jnp.float32),
        jax.ShapeDtypeStruct((NPAD, W), jnp.float32),
    ),
)

_combine_tc = pl.pallas_call(
    _combine_body,
    out_shape=jax.ShapeDtypeStruct((NPAD, W), jnp.float32),
)

_final_tc = pl.pallas_call(
    _final_body,
    out_shape=jax.ShapeDtypeStruct((NPAD, D_OUT), jnp.float32),
)


def kernel(x, edge_index, W1, b1, W2, b2):
    x = x.astype(jnp.float32)
    src = edge_index[0]
    dst = edge_index[1]
    npad_e = EPAD - E
    pad_ids = (N + (jnp.arange(npad_e, dtype=jnp.int32) % (NPAD - N)))
    src2d = jnp.concatenate([src, pad_ids]).reshape(EROWS, IW)
    dst2d = jnp.concatenate([dst, pad_ids]).reshape(EROWS, IW)

    xp = jnp.pad(x, ((0, NPAD - N), (0, 0)))
    w2p = jnp.pad(W2, ((0, 0), (0, W - D_OUT)))
    b2p = jnp.pad(b2, ((0, W - D_OUT),)).reshape(1, W)
    dacc = _deg_sc(dst2d)
    h0 = _mlp_tc(xp, W1, b1.reshape(1, D_H), w2p, b2p)
    g0, cb, sd = _prep_tc(h0, dacc)

    def _round(_, g):
        acc = _prop_sc(g, src2d, dst2d)
        return _combine_tc(acc.reshape(NPAD, W), cb, g0)

    g = lax.fori_loop(0, K, _round, g0)
    out = _final_tc(g, sd)
    return out[:N]
